# baseline (speedup 1.0000x reference)
import functools

import jax
import jax.numpy as jnp
from jax.experimental import pallas as pl

_SPARQ = 32
_TOP_K = 512


def _ordered_u32(x):
    u = jax.lax.bitcast_convert_type(x, jnp.uint32)
    flip = jnp.where(u >= jnp.uint32(0x80000000),
                     jnp.uint32(0xFFFFFFFF), jnp.uint32(0x80000000))
    return u ^ flip


def _kth_largest_u32(o, kk, nbits=32):
    R = o.shape[0]
    p = jnp.zeros((R, 1), jnp.uint32)
    for b in range(nbits - 1, -1, -1):
        cand = p | jnp.uint32(1 << b)
        cnt = jnp.sum((o >= cand).astype(jnp.int32), axis=1, keepdims=True)
        p = jnp.where(cnt >= kk, cand, p)
    return p


def _body(q_ref, k_ref, v_ref, am_ref, o_ref, *, R, E, kk, t0, do_select):
    qt = q_ref[0]
    kt = k_ref[0]
    vt = v_ref[0]
    am = am_ref[...]

    rows = t0 + jax.lax.broadcasted_iota(jnp.int32, (R, E), 0)
    cols = jax.lax.broadcasted_iota(jnp.int32, (R, E), 1)
    valid = (cols <= rows) & (am > jnp.float32(0.5))

    scores = jax.lax.dot_general(qt, kt, (((1,), (1,)), ((), ())),
                                 preferred_element_type=jnp.float32)

    if do_select:
        aq = jnp.abs(qt)
        ai = jax.lax.bitcast_convert_type(aq, jnp.uint32)
        pq = _kth_largest_u32(ai, _SPARQ, nbits=31)
        qs = jnp.where(ai >= pq, qt, jnp.float32(0.0))

        approx = jax.lax.dot_general(qs, kt, (((1,), (1,)), ((), ())),
                                     preferred_element_type=jnp.float32)
        ax = jnp.where(valid, approx, jnp.float32(-jnp.inf))

        o = _ordered_u32(ax)
        pth = _kth_largest_u32(o, kk)
        sel = (o >= pth) & valid
    else:
        sel = valid

    s = jnp.where(sel, scores, jnp.float32(-32000.0))
    m = jnp.max(s, axis=1, keepdims=True)
    e = jnp.exp(s - m)
    num = jax.lax.dot_general(e, vt, (((1,), (0,)), ((), ())),
                              preferred_element_type=jnp.float32)
    denom = jnp.sum(e, axis=1, keepdims=True)
    o_ref[0] = num / denom


def kernel(q, k, v, attention_mask):
    N, H, T, HID = q.shape
    B = N * H
    R = min(128, T)
    nt = T // R
    kk = min(_TOP_K, T)
    qf = q.reshape(B, T, HID)
    kf = k.reshape(B, T, HID)
    vf = v.reshape(B, T, HID)

    pieces = []
    for ti in range(nt):
        E = (ti + 1) * R
        pieces.append(pl.pallas_call(
            functools.partial(_body, R=R, E=E, kk=kk, t0=ti * R,
                              do_select=(E > kk)),
            grid=(B,),
            in_specs=[
                pl.BlockSpec((1, R, HID), lambda b, ti=ti: (b, ti, 0)),
                pl.BlockSpec((1, E, HID), lambda b: (b, 0, 0)),
                pl.BlockSpec((1, E, HID), lambda b: (b, 0, 0)),
                pl.BlockSpec((1, E), lambda b: (b // H, 0)),
            ],
            out_specs=pl.BlockSpec((1, R, HID), lambda b: (b, 0, 0)),
            out_shape=jax.ShapeDtypeStruct((B, R, HID), jnp.float32),
        )(qf, kf, vf, attention_mask))
    out = jnp.concatenate(pieces, axis=1)
    return out.reshape(N, H, T, HID)


# speedup vs baseline: 1.3626x; 1.3626x over previous
import functools

import jax
import jax.numpy as jnp
from jax.experimental import pallas as pl

_SPARQ = 32
_TOP_K = 512


def _ordered_u32(x):
    u = jax.lax.bitcast_convert_type(x, jnp.uint32)
    flip = jnp.where(u >= jnp.uint32(0x80000000),
                     jnp.uint32(0xFFFFFFFF), jnp.uint32(0x80000000))
    return u ^ flip


def _kth_largest_u32(o, kk, nbits=32):
    R = o.shape[0]
    p = jnp.zeros((R, 1), jnp.uint32)
    for b in range(nbits - 1, -1, -1):
        cand = p | jnp.uint32(1 << b)
        cnt = jnp.sum((o >= cand).astype(jnp.int32), axis=1, keepdims=True)
        p = jnp.where(cnt >= kk, cand, p)
    return p


def _body(q_ref, k_ref, v_ref, am_ref, o_ref, *, R, E, kk, t0, do_select):
    qt = q_ref[0]
    kt = k_ref[0]
    vt = v_ref[0]
    am = am_ref[...]

    if t0 is None:
        t0 = pl.program_id(1) * R
    rows = t0 + jax.lax.broadcasted_iota(jnp.int32, (R, E), 0)
    cols = jax.lax.broadcasted_iota(jnp.int32, (R, E), 1)
    valid = (cols <= rows) & (am > jnp.float32(0.5))

    scores = jax.lax.dot_general(qt, kt, (((1,), (1,)), ((), ())),
                                 preferred_element_type=jnp.float32)

    if do_select:
        aq = jnp.abs(qt)
        ai = jax.lax.bitcast_convert_type(aq, jnp.uint32)
        pq = _kth_largest_u32(ai, _SPARQ, nbits=31)
        qs = jnp.where(ai >= pq, qt, jnp.float32(0.0))

        approx = jax.lax.dot_general(qs, kt, (((1,), (1,)), ((), ())),
                                     preferred_element_type=jnp.float32)
        ax = jnp.where(valid, approx, jnp.float32(-jnp.inf))

        o = _ordered_u32(ax)
        pth = _kth_largest_u32(o, kk)
        sel = o >= pth
    else:
        sel = valid

    s = jnp.where(sel, scores, jnp.float32(-32000.0))
    m = jnp.max(s, axis=1, keepdims=True)
    e = jnp.exp(s - m)
    num = jax.lax.dot_general(e, vt, (((1,), (0,)), ((), ())),
                              preferred_element_type=jnp.float32)
    denom = jnp.sum(e, axis=1, keepdims=True)
    o_ref[0] = num / denom


def kernel(q, k, v, attention_mask):
    N, H, T, HID = q.shape
    B = N * H
    R = min(256, T)
    nt = T // R
    kk = min(_TOP_K, T)
    qf = q.reshape(B, T, HID)
    kf = k.reshape(B, T, HID)
    vf = v.reshape(B, T, HID)

    pieces = []
    nt0 = max(1, min(nt, kk // R))
    E0 = nt0 * R
    pieces.append(pl.pallas_call(
        functools.partial(_body, R=R, E=E0, kk=kk, t0=None, do_select=False),
        grid=(B, nt0),
        in_specs=[
            pl.BlockSpec((1, R, HID), lambda b, t: (b, t, 0)),
            pl.BlockSpec((1, E0, HID), lambda b, t: (b, 0, 0)),
            pl.BlockSpec((1, E0, HID), lambda b, t: (b, 0, 0)),
            pl.BlockSpec((1, E0), lambda b, t: (b // H, 0)),
        ],
        out_specs=pl.BlockSpec((1, R, HID), lambda b, t: (b, t, 0)),
        out_shape=jax.ShapeDtypeStruct((B, E0, HID), jnp.float32),
    )(qf, kf, vf, attention_mask))
    for ti in range(nt0, nt):
        E = (ti + 1) * R
        pieces.append(pl.pallas_call(
            functools.partial(_body, R=R, E=E, kk=kk, t0=ti * R,
                              do_select=True),
            grid=(B,),
            in_specs=[
                pl.BlockSpec((1, R, HID), lambda b, ti=ti: (b, ti, 0)),
                pl.BlockSpec((1, E, HID), lambda b: (b, 0, 0)),
                pl.BlockSpec((1, E, HID), lambda b: (b, 0, 0)),
                pl.BlockSpec((1, E), lambda b: (b // H, 0)),
            ],
            out_specs=pl.BlockSpec((1, R, HID), lambda b: (b, 0, 0)),
            out_shape=jax.ShapeDtypeStruct((B, R, HID), jnp.float32),
        )(qf, kf, vf, attention_mask))
    out = jnp.concatenate(pieces, axis=1)
    return out.reshape(N, H, T, HID)


# speedup vs baseline: 1.7041x; 1.2506x over previous
import functools

import jax
import jax.numpy as jnp
from jax.experimental import pallas as pl

_SPARQ = 32
_TOP_K = 512


def _ordered_u32(x):
    u = jax.lax.bitcast_convert_type(x, jnp.uint32)
    flip = jnp.where(u >= jnp.uint32(0x80000000),
                     jnp.uint32(0xFFFFFFFF), jnp.uint32(0x80000000))
    return u ^ flip


def _ordered_inv(o):
    flip = jnp.where(o >= jnp.uint32(0x80000000),
                     jnp.uint32(0x80000000), jnp.uint32(0xFFFFFFFF))
    return jax.lax.bitcast_convert_type(o ^ flip, jnp.float32)


def _kth_largest_ax0(o, kk, nbits=32):
    S, R = o.shape
    ch = max(1, S // 256)
    o3 = o.reshape(ch, S // ch, R)
    p = jnp.zeros((1, R), jnp.uint32)
    for b in range(nbits - 1, -1, -1):
        cand = p | jnp.uint32(1 << b)
        w = (o3 >= cand).astype(jnp.int32)
        part = jnp.sum(w, axis=1)
        cnt = jnp.sum(part, axis=0, keepdims=True)
        p = jnp.where(cnt >= kk, cand, p)
    return p


def _body(qT_ref, q_ref, k_ref, v_ref, amT_ref, o_ref, *, R, E, kk, t0,
          do_select):
    qt = q_ref[0]
    kt = k_ref[0]
    vt = v_ref[0]
    amT = amT_ref[...]

    if t0 is None:
        t0 = pl.program_id(1) * R
    rows = t0 + jax.lax.broadcasted_iota(jnp.int32, (R, E), 0)
    cols = jax.lax.broadcasted_iota(jnp.int32, (R, E), 1)
    amrow = jnp.transpose(amT)
    valid = (cols <= rows) & (amrow > jnp.float32(0.5))

    scores = jax.lax.dot_general(qt, kt, (((1,), (1,)), ((), ())),
                                 preferred_element_type=jnp.float32)

    if do_select:
        qT = qT_ref[0]
        aiT = jax.lax.bitcast_convert_type(jnp.abs(qT), jnp.uint32)
        pqT = _kth_largest_ax0(aiT, _SPARQ, nbits=31)
        qsT = jnp.where(aiT >= pqT, qT, jnp.float32(0.0))
        ai = jax.lax.bitcast_convert_type(jnp.abs(qt), jnp.uint32)
        qs = jnp.where(ai >= jnp.transpose(pqT), qt, jnp.float32(0.0))

        approxT = jax.lax.dot_general(kt, qsT, (((1,), (0,)), ((), ())),
                                      preferred_element_type=jnp.float32)
        rowsT = t0 + jax.lax.broadcasted_iota(jnp.int32, (E, R), 1)
        colsT = jax.lax.broadcasted_iota(jnp.int32, (E, R), 0)
        validT = (colsT <= rowsT) & (amT > jnp.float32(0.5))
        axT = jnp.where(validT, approxT, jnp.float32(-jnp.inf))
        pthT = _kth_largest_ax0(_ordered_u32(axT), kk)

        tau = jnp.transpose(_ordered_inv(pthT))
        approx = jax.lax.dot_general(qs, kt, (((1,), (1,)), ((), ())),
                                     preferred_element_type=jnp.float32)
        sel = (approx >= tau) & valid
    else:
        sel = valid

    s = jnp.where(sel, scores, jnp.float32(-32000.0))
    m = jnp.max(s, axis=1, keepdims=True)
    e = jnp.exp(s - m)
    num = jax.lax.dot_general(e, vt, (((1,), (0,)), ((), ())),
                              preferred_element_type=jnp.float32)
    denom = jnp.sum(e, axis=1, keepdims=True)
    o_ref[0] = num / denom


def kernel(q, k, v, attention_mask):
    N, H, T, HID = q.shape
    B = N * H
    R = min(256, T)
    nt = T // R
    kk = min(_TOP_K, T)
    qf = q.reshape(B, T, HID)
    kf = k.reshape(B, T, HID)
    vf = v.reshape(B, T, HID)
    qTf = jnp.transpose(qf, (0, 2, 1))
    amT = jnp.transpose(attention_mask)

    pieces = []
    nt0 = max(1, min(nt, kk // R))
    E0 = nt0 * R
    pieces.append(pl.pallas_call(
        functools.partial(_body, R=R, E=E0, kk=kk, t0=None, do_select=False),
        grid=(B, nt0),
        in_specs=[
            pl.BlockSpec((1, HID, R), lambda b, t: (b, 0, t)),
            pl.BlockSpec((1, R, HID), lambda b, t: (b, t, 0)),
            pl.BlockSpec((1, E0, HID), lambda b, t: (b, 0, 0)),
            pl.BlockSpec((1, E0, HID), lambda b, t: (b, 0, 0)),
            pl.BlockSpec((E0, 1), lambda b, t: (0, b // H)),
        ],
        out_specs=pl.BlockSpec((1, R, HID), lambda b, t: (b, t, 0)),
        out_shape=jax.ShapeDtypeStruct((B, E0, HID), jnp.float32),
    )(qTf, qf, kf, vf, amT))
    for ti in range(nt0, nt):
        E = (ti + 1) * R
        pieces.append(pl.pallas_call(
            functools.partial(_body, R=R, E=E, kk=kk, t0=ti * R,
                              do_select=True),
            grid=(B,),
            in_specs=[
                pl.BlockSpec((1, HID, R), lambda b, ti=ti: (b, 0, ti)),
                pl.BlockSpec((1, R, HID), lambda b, ti=ti: (b, ti, 0)),
                pl.BlockSpec((1, E, HID), lambda b: (b, 0, 0)),
                pl.BlockSpec((1, E, HID), lambda b: (b, 0, 0)),
                pl.BlockSpec((E, 1), lambda b: (0, b // H)),
            ],
            out_specs=pl.BlockSpec((1, R, HID), lambda b: (b, 0, 0)),
            out_shape=jax.ShapeDtypeStruct((B, R, HID), jnp.float32),
        )(qTf, qf, kf, vf, amT))
    out = jnp.concatenate(pieces, axis=1)
    return out.reshape(N, H, T, HID)


# speedup vs baseline: 1.8008x; 1.0568x over previous
import functools

import jax
import jax.numpy as jnp
from jax.experimental import pallas as pl

_SPARQ = 32
_TOP_K = 512


def _ordered_u32(x):
    u = jax.lax.bitcast_convert_type(x, jnp.uint32)
    flip = jnp.where(u >= jnp.uint32(0x80000000),
                     jnp.uint32(0xFFFFFFFF), jnp.uint32(0x80000000))
    return u ^ flip


def _ordered_inv(o):
    flip = jnp.where(o >= jnp.uint32(0x80000000),
                     jnp.uint32(0x80000000), jnp.uint32(0xFFFFFFFF))
    return jax.lax.bitcast_convert_type(o ^ flip, jnp.float32)


def _kth_largest_ax0(o, kk, nbits=32):
    S, R = o.shape
    ch = max(1, S // 256)
    o3 = o.reshape(ch, S // ch, R)
    p = jnp.zeros((1, R), jnp.uint32)
    for b in range(nbits - 1, -1, -1):
        cand = p | jnp.uint32(1 << b)
        w = (o3 >= cand).astype(jnp.int32)
        part = jnp.sum(w, axis=1)
        cnt = jnp.sum(part, axis=0, keepdims=True)
        p = jnp.where(cnt >= kk, cand, p)
    return p


def _body(qT_ref, q_ref, k_ref, v_ref, amT_ref, o_ref, *, R, E, kk, t0,
          do_select):
    qt = q_ref[0]
    kt = k_ref[0]
    vt = v_ref[0]
    amT = amT_ref[...]

    if t0 is None:
        t0 = pl.program_id(1) * R
    rows = t0 + jax.lax.broadcasted_iota(jnp.int32, (R, E), 0)
    cols = jax.lax.broadcasted_iota(jnp.int32, (R, E), 1)
    amrow = jnp.transpose(amT)
    valid = (cols <= rows) & (amrow > jnp.float32(0.5))

    scores = jax.lax.dot_general(qt, kt, (((1,), (1,)), ((), ())),
                                 preferred_element_type=jnp.float32)

    if do_select:
        qT = qT_ref[0]
        aiT = jax.lax.bitcast_convert_type(jnp.abs(qT), jnp.uint32)
        pqT = _kth_largest_ax0(aiT, _SPARQ, nbits=31)
        qsT = jnp.where(aiT >= pqT, qT, jnp.float32(0.0))
        ai = jax.lax.bitcast_convert_type(jnp.abs(qt), jnp.uint32)
        qs = jnp.where(ai >= jnp.transpose(pqT), qt, jnp.float32(0.0))

        approxT = jax.lax.dot_general(kt, qsT, (((1,), (0,)), ((), ())),
                                      preferred_element_type=jnp.float32)
        rowsT = t0 + jax.lax.broadcasted_iota(jnp.int32, (E, R), 1)
        colsT = jax.lax.broadcasted_iota(jnp.int32, (E, R), 0)
        validT = (colsT <= rowsT) & (amT > jnp.float32(0.5))
        axT = jnp.where(validT, approxT, jnp.float32(-jnp.inf))
        pthT = _kth_largest_ax0(_ordered_u32(axT), kk)

        tau = jnp.transpose(_ordered_inv(pthT))
        approx = jax.lax.dot_general(qs, kt, (((1,), (1,)), ((), ())),
                                     preferred_element_type=jnp.float32)
        sel = (approx >= tau) & valid
    else:
        sel = valid

    s = jnp.where(sel, scores, jnp.float32(-32000.0))
    m = jnp.max(s, axis=1, keepdims=True)
    e = jnp.exp(s - m)
    num = jax.lax.dot_general(e, vt, (((1,), (0,)), ((), ())),
                              preferred_element_type=jnp.float32)
    denom = jnp.sum(e, axis=1, keepdims=True)
    o_ref[0] = num / denom


def kernel(q, k, v, attention_mask):
    N, H, T, HID = q.shape
    B = N * H
    R = min(512, T)
    nt = T // R
    kk = min(_TOP_K, T)
    qf = q.reshape(B, T, HID)
    kf = k.reshape(B, T, HID)
    vf = v.reshape(B, T, HID)
    qTf = jnp.transpose(qf, (0, 2, 1))
    amT = jnp.transpose(attention_mask)

    pieces = []
    nt0 = max(1, min(nt, kk // R))
    E0 = nt0 * R
    pieces.append(pl.pallas_call(
        functools.partial(_body, R=R, E=E0, kk=kk, t0=None, do_select=False),
        grid=(B, nt0),
        in_specs=[
            pl.BlockSpec((1, HID, R), lambda b, t: (b, 0, t)),
            pl.BlockSpec((1, R, HID), lambda b, t: (b, t, 0)),
            pl.BlockSpec((1, E0, HID), lambda b, t: (b, 0, 0)),
            pl.BlockSpec((1, E0, HID), lambda b, t: (b, 0, 0)),
            pl.BlockSpec((E0, 1), lambda b, t: (0, b // H)),
        ],
        out_specs=pl.BlockSpec((1, R, HID), lambda b, t: (b, t, 0)),
        out_shape=jax.ShapeDtypeStruct((B, E0, HID), jnp.float32),
    )(qTf, qf, kf, vf, amT))
    for ti in range(nt0, nt):
        E = (ti + 1) * R
        pieces.append(pl.pallas_call(
            functools.partial(_body, R=R, E=E, kk=kk, t0=ti * R,
                              do_select=True),
            grid=(B,),
            in_specs=[
                pl.BlockSpec((1, HID, R), lambda b, ti=ti: (b, 0, ti)),
                pl.BlockSpec((1, R, HID), lambda b, ti=ti: (b, ti, 0)),
                pl.BlockSpec((1, E, HID), lambda b: (b, 0, 0)),
                pl.BlockSpec((1, E, HID), lambda b: (b, 0, 0)),
                pl.BlockSpec((E, 1), lambda b: (0, b // H)),
            ],
            out_specs=pl.BlockSpec((1, R, HID), lambda b: (b, 0, 0)),
            out_shape=jax.ShapeDtypeStruct((B, R, HID), jnp.float32),
        )(qTf, qf, kf, vf, amT))
    out = jnp.concatenate(pieces, axis=1)
    return out.reshape(N, H, T, HID)
